# Initial kernel scaffold; baseline (speedup 1.0000x reference)
import jax
import jax.numpy as jnp
from jax.experimental import pallas as pl


def kernel(hn, he, edge_index, fe, fes, norm, ev_W1, ev_b1, ev_W2, ev_b2, fc_W1, fc_W2, nu_W1, nu_b1, nu_W2, nu_b2):
    raise NotImplementedError("write your pallas kernel here")


# speedup vs baseline: 1.5455x; 1.5455x over previous
import functools
import jax
import jax.numpy as jnp
from jax import lax
from jax.experimental import pallas as pl
from jax.experimental.pallas import tpu as pltpu
from jax.experimental.pallas import tpu_sc as plsc

N_NODES = 10000
E = 160000
D = 128
D_VAL = 16
NUM_FES = 16
H1 = 512
H_FC = 64
CHUNK = 128
NCHUNKS = E // CHUNK
NC, NS = 2, 16
NW = NC * NS
ITERS = (NCHUNKS + NW - 1) // NW
NR_CHUNK = 80
NRCHUNKS = N_NODES // NR_CHUNK
NR_ITERS = (NRCHUNKS + NS - 1) // NS

_mesh = plsc.VectorSubcoreMesh(core_axis_name="c", subcore_axis_name="s")


def _gather_body(hn_hbm, src_hbm, dst_hbm, hns_hbm, hnd_hbm,
                 idx_s, idx_d, rows_s, rows_d, sem):
    cid = lax.axis_index("c")
    sid = lax.axis_index("s")
    wid = sid * NC + cid

    def body(i, carry):
        c = wid + i * NW

        @pl.when(c < NCHUNKS)
        def _():
            base = c * CHUNK
            pltpu.sync_copy(src_hbm.at[pl.ds(base, CHUNK)], idx_s)
            pltpu.sync_copy(dst_hbm.at[pl.ds(base, CHUNK)], idx_d)
            ca = pltpu.async_copy(hn_hbm.at[idx_s], rows_s, sem)
            cb = pltpu.async_copy(hn_hbm.at[idx_d], rows_d, sem)
            ca.wait()
            cb.wait()
            pltpu.sync_copy(rows_s, hns_hbm.at[pl.ds(base, CHUNK)])
            pltpu.sync_copy(rows_d, hnd_hbm.at[pl.ds(base, CHUNK)])

        return carry

    lax.fori_loop(0, ITERS, body, 0)


_gather = pl.kernel(
    _gather_body,
    mesh=_mesh,
    out_type=[jax.ShapeDtypeStruct((E, D), jnp.float32),
              jax.ShapeDtypeStruct((E, D), jnp.float32)],
    scratch_types=[
        pltpu.VMEM((CHUNK,), jnp.int32),
        pltpu.VMEM((CHUNK,), jnp.int32),
        pltpu.VMEM((CHUNK, D), jnp.float32),
        pltpu.VMEM((CHUNK, D), jnp.float32),
        pltpu.SemaphoreType.DMA,
    ],
)


def _scatter_body(henw_hbm, dst_hbm, zeros_hbm, out_hbm, idx2, rows, acc):
    cid = lax.axis_index("c")
    sid = lax.axis_index("s")
    wid = sid * NC + cid

    def zbody(i, carry):
        c = sid + i * NS

        @pl.when(c < NRCHUNKS)
        def _():
            pltpu.sync_copy(zeros_hbm, acc.at[pl.ds(c * NR_CHUNK, NR_CHUNK)])

        return carry

    lax.fori_loop(0, NR_ITERS, zbody, 0)
    plsc.subcore_barrier()

    def body(i, carry):
        c = wid + i * NW

        @pl.when(c < NCHUNKS)
        def _():
            base = c * CHUNK
            pltpu.sync_copy(dst_hbm.at[pl.ds(base, CHUNK)], idx2.at[0])
            pltpu.sync_copy(henw_hbm.at[pl.ds(base, CHUNK)], rows)
            pltpu.sync_copy(rows, acc.at[idx2.at[0]], add=True)

        return carry

    lax.fori_loop(0, ITERS, body, 0)
    plsc.subcore_barrier()

    def dbody(i, carry):
        c = sid + i * NS

        @pl.when(c < NRCHUNKS)
        def _():
            pltpu.sync_copy(acc.at[pl.ds(c * NR_CHUNK, NR_CHUNK)],
                            out_hbm.at[cid, pl.ds(c * NR_CHUNK, NR_CHUNK)])

        return carry

    lax.fori_loop(0, NR_ITERS, dbody, 0)


_scatter = pl.kernel(
    _scatter_body,
    mesh=_mesh,
    out_type=jax.ShapeDtypeStruct((NC, N_NODES, D), jnp.float32),
    scratch_types=[
        pltpu.VMEM((1, CHUNK), jnp.int32),
        pltpu.VMEM((CHUNK, D), jnp.float32),
        pltpu.VMEM_SHARED((N_NODES, D), jnp.float32),
    ],
)


BE = 640


def _edge_body(he_r, hns_r, hnd_r, fes_r, fn_r,
               w1a_r, w1b_r, w1c_r, b1_r, w2_r, b2_r, fw1_r, fw2_r,
               hen_r, henw_r):
    t = jnp.dot(he_r[:], w1a_r[:], preferred_element_type=jnp.float32)
    t = t + jnp.dot(hns_r[:], w1b_r[:], preferred_element_type=jnp.float32)
    t = t + jnp.dot(hnd_r[:], w1c_r[:], preferred_element_type=jnp.float32)
    t = jnp.maximum(t + b1_r[:], 0.0)
    v = jnp.dot(t, w2_r[:], preferred_element_type=jnp.float32) + b2_r[:]
    h = jnp.maximum(
        jnp.dot(fes_r[:], fw1_r[:], preferred_element_type=jnp.float32) * 0.25,
        0.0)
    a = jnp.dot(h, fw2_r[:], preferred_element_type=jnp.float32)
    heu = v[:, 0:1] * a[:, 0:D]
    for i in range(1, D_VAL):
        heu = heu + v[:, i:i + 1] * a[:, i * D:(i + 1) * D]
    hen = he_r[:] + heu * (fn_r[:, 0:1] * (1.0 / 32.0))
    hen_r[:] = hen
    henw_r[:] = hen * fn_r[:, 1:2]


def _edge_call(he, hns, hnd, fes, fn, w1a, w1b, w1c, b1, w2, b2, fw1, fw2):
    blk = lambda r, c: pl.BlockSpec((r, c), lambda i: (i, 0))
    full = lambda r, c: pl.BlockSpec((r, c), lambda i: (0, 0))
    return pl.pallas_call(
        _edge_body,
        grid=(E // BE,),
        in_specs=[
            blk(BE, D), blk(BE, D), blk(BE, D), blk(BE, NUM_FES), blk(BE, 2),
            full(D, H1), full(D, H1), full(D, H1), full(1, H1),
            full(H1, D_VAL), full(1, D_VAL),
            full(NUM_FES, H_FC), full(H_FC, D_VAL * D),
        ],
        out_specs=[blk(BE, D), blk(BE, D)],
        out_shape=[jax.ShapeDtypeStruct((E, D), jnp.float32),
                   jax.ShapeDtypeStruct((E, D), jnp.float32)],
    )(he, hns, hnd, fes, fn, w1a, w1b, w1c, b1, w2, b2, fw1, fw2)


BN = 1000


def _node_body(hn_r, pr_r, w1a_r, w1b_r, b1_r, w2_r, b2_r, hnn_r):
    nt = pr_r[0] + pr_r[1]
    u = jnp.dot(hn_r[:], w1a_r[:], preferred_element_type=jnp.float32)
    u = u + jnp.dot(nt, w1b_r[:], preferred_element_type=jnp.float32)
    u = jnp.maximum(u + b1_r[:], 0.0)
    hnn_r[:] = hn_r[:] + jnp.dot(u, w2_r[:],
                                 preferred_element_type=jnp.float32) + b2_r[:]


def _node_call(hn, partials, w1a, w1b, b1, w2, b2):
    return pl.pallas_call(
        _node_body,
        grid=(N_NODES // BN,),
        in_specs=[
            pl.BlockSpec((BN, D), lambda i: (i, 0)),
            pl.BlockSpec((NC, BN, D), lambda i: (0, i, 0)),
            pl.BlockSpec((D, H1), lambda i: (0, 0)),
            pl.BlockSpec((D, H1), lambda i: (0, 0)),
            pl.BlockSpec((1, H1), lambda i: (0, 0)),
            pl.BlockSpec((H1, D), lambda i: (0, 0)),
            pl.BlockSpec((1, D), lambda i: (0, 0)),
        ],
        out_specs=pl.BlockSpec((BN, D), lambda i: (i, 0)),
        out_shape=jax.ShapeDtypeStruct((N_NODES, D), jnp.float32),
    )(hn, partials, w1a, w1b, b1, w2, b2)


@jax.jit
def kernel(hn, he, edge_index, fe, fes, norm,
           ev_W1, ev_b1, ev_W2, ev_b2, fc_W1, fc_W2,
           nu_W1, nu_b1, nu_W2, nu_b2):
    src = edge_index[0]
    dst = edge_index[1]
    hns, hnd = _gather(hn, src, dst)
    fn = jnp.concatenate([fe, norm[:, None]], axis=1)
    hen, henw = _edge_call(
        he, hns, hnd, fes, fn,
        ev_W1[:D], ev_W1[D:2 * D], ev_W1[2 * D:],
        ev_b1.reshape(1, H1), ev_W2, ev_b2.reshape(1, D_VAL),
        fc_W1, fc_W2)
    partials = _scatter(henw, dst, jnp.zeros((NR_CHUNK, D), jnp.float32))
    hnn = _node_call(hn, partials,
                     nu_W1[:D], nu_W1[D:], nu_b1.reshape(1, H1),
                     nu_W2, nu_b2.reshape(1, D))
    return hnn, hen
